# baseline (speedup 1.0000x reference)
import functools

import jax
import jax.numpy as jnp
from jax import lax
from jax.experimental import pallas as pl
from jax.experimental.pallas import tpu as pltpu
from jax.experimental.pallas import tpu_sc as plsc

_B = 16384
_D = 64
_DR = 32
_NC, _NS = 2, 16
_NW = _NC * _NS
_BPW = _B // _NW
_CH = 128
_NCH = _BPW // _CH

_GAMMA = 12.0
_EPSILON = 2.0
_EMB_RANGE = (_GAMMA + _EPSILON) / _D
_PI = 3.141592653589793

@functools.cache
def _build_sc_gather():
    mesh = plsc.VectorSubcoreMesh(core_axis_name="c", subcore_axis_name="s")

    @functools.partial(
        pl.kernel,
        out_type=[
            jax.ShapeDtypeStruct((_B, _D), jnp.float32),
            jax.ShapeDtypeStruct((_B, _D), jnp.float32),
            jax.ShapeDtypeStruct((_B, _DR), jnp.float32),
        ],
        mesh=mesh,
        compiler_params=pltpu.CompilerParams(use_tc_tiling_on_sc=False),
        scratch_types=[
            pltpu.VMEM((_NCH, _CH), jnp.int32),
            pltpu.VMEM((_NCH, _CH), jnp.int32),
            pltpu.VMEM((_NCH, _CH), jnp.int32),
            pltpu.VMEM((_BPW, _D), jnp.float32),
            pltpu.VMEM((_BPW, _D), jnp.float32),
            pltpu.VMEM((_BPW, _DR), jnp.float32),
            pltpu.SemaphoreType.DMA,
        ],
    )
    def sc_gather(ent_hbm, rel_hbm, hidx_hbm, ridx_hbm, tidx_hbm,
                  head_out, tail_out, rel_out,
                  hidx_v, ridx_v, tidx_v, head_v, tail_v, rel_v, sem):
        wid = lax.axis_index("s") * _NC + lax.axis_index("c")
        base = wid * _BPW
        pltpu.sync_copy(hidx_hbm.at[wid], hidx_v)
        pltpu.sync_copy(ridx_hbm.at[wid], ridx_v)
        pltpu.sync_copy(tidx_hbm.at[wid], tidx_v)
        copies = []
        for j in range(_NCH):
            sl = pl.ds(j * _CH, _CH)
            copies.append(pltpu.async_copy(ent_hbm.at[hidx_v.at[j]], head_v.at[sl], sem))
            copies.append(pltpu.async_copy(ent_hbm.at[tidx_v.at[j]], tail_v.at[sl], sem))
            copies.append(pltpu.async_copy(rel_hbm.at[ridx_v.at[j]], rel_v.at[sl], sem))
        for c in copies:
            c.wait()
        pltpu.sync_copy(head_v, head_out.at[pl.ds(base, _BPW)])
        pltpu.sync_copy(tail_v, tail_out.at[pl.ds(base, _BPW)])
        pltpu.sync_copy(rel_v, rel_out.at[pl.ds(base, _BPW)])

    return sc_gather


def _score_body(head_ref, tail_ref, rel_ref, out_ref):
    head = head_ref[...]
    tail = tail_ref[...]
    rel = rel_ref[...]
    re_h, im_h = head[:, :_DR], head[:, _DR:]
    re_t, im_t = tail[:, :_DR], tail[:, _DR:]
    phase = rel * (_PI / _EMB_RANGE)
    re_r = jnp.cos(phase)
    im_r = jnp.sin(phase)
    re_s = re_r * re_t + im_r * im_t - re_h
    im_s = re_r * im_t - im_r * re_t - im_h
    score = jnp.sqrt(re_s * re_s + im_s * im_s)
    out_ref[...] = jnp.sum(score, axis=1, keepdims=True)


_BLK = 2048

_score = pl.pallas_call(
    _score_body,
    grid=(_B // _BLK,),
    in_specs=[
        pl.BlockSpec((_BLK, _D), lambda i: (i, 0)),
        pl.BlockSpec((_BLK, _D), lambda i: (i, 0)),
        pl.BlockSpec((_BLK, _DR), lambda i: (i, 0)),
    ],
    out_specs=pl.BlockSpec((_BLK, 1), lambda i: (i, 0)),
    out_shape=jax.ShapeDtypeStruct((_B, 1), jnp.float32),
)


def kernel(sample, entity_embedding, relation_embedding):
    hidx = sample[:, 0].reshape(_NW, _NCH, _CH)
    ridx = sample[:, 1].reshape(_NW, _NCH, _CH)
    tidx = sample[:, 2].reshape(_NW, _NCH, _CH)
    head, tail, rel = _build_sc_gather()(
        entity_embedding, relation_embedding, hidx, ridx, tidx)
    return _score(head, tail, rel)
